# baseline (speedup 1.0000x reference)
import functools
import math

import jax
import jax.numpy as jnp
from jax.experimental import pallas as pl
from jax.experimental.pallas import tpu as pltpu

_BASE_GSD = 10.0
_D = 192
_LN1E4_OVER = math.log(10000.0) / (_D // 4)


def _tc_body(months_ref, gsd_ref, pos_ref, mtab_ref, ch7_ref, ch_sp_ref,
             ch_t_ref, ch_st_ref, s_t_ref, sp_ref, t_ref, st_ref,
             s_t_out_ref, sp_out_ref, t_out_ref, st_out_ref, spat_pad_ref):
    b = pl.program_id(0)
    t = pl.program_id(1)
    h, w = 8, 8
    d = _D

    @pl.when(t == 0)
    def _spatial_and_small():
        res = gsd_ref[0]
        wc = jax.lax.broadcasted_iota(jnp.int32, (h, w, d), 1).astype(jnp.float32)
        hc = jax.lax.broadcasted_iota(jnp.int32, (h, w, d), 0).astype(jnp.float32)
        col = jax.lax.broadcasted_iota(jnp.int32, (h, w, d), 2)
        p = jnp.where(col < d // 2, wc, hc) * res
        k = col % (d // 2)
        kk = (k % (d // 4)).astype(jnp.float32)
        omega = jnp.exp(kk * (-_LN1E4_OVER))
        phase = jnp.where(k < d // 4, 0.0, 0.5 * jnp.pi).astype(jnp.float32)
        spatial = jnp.sin(p * omega + phase)
        spat_pad_ref[...] = jnp.concatenate(
            [jnp.zeros((h, w, 3 * d), jnp.float32), spatial], axis=-1)

        sp_row = jnp.concatenate(
            [ch_sp_ref[...], jnp.zeros((3, 3 * d), jnp.float32)], axis=-1)
        sp_out_ref[0] = (sp_ref[0] + sp_row[None, None]
                         + spat_pad_ref[...][:, :, None, :])

        st_row = jnp.concatenate(
            [ch_st_ref[...], jnp.zeros((3, 3 * d), jnp.float32)], axis=-1)
        st_out_ref[0] = st_ref[0] + st_row

    pos_row = pos_ref[pl.ds(t, 1), :]
    m = months_ref[b, t]
    mon_row = mtab_ref[pl.ds(m, 1), :]

    emb_t = jnp.concatenate(
        [ch_t_ref[...],
         jnp.broadcast_to(pos_row, (3, d)),
         jnp.broadcast_to(mon_row, (3, d)),
         jnp.zeros((3, d), jnp.float32)], axis=-1)
    t_out_ref[0, 0] = t_ref[0, 0] + emb_t

    emb7 = jnp.concatenate(
        [ch7_ref[...],
         jnp.broadcast_to(pos_row, (7, d)),
         jnp.broadcast_to(mon_row, (7, d)),
         jnp.zeros((7, d), jnp.float32)], axis=-1)
    s_t_out_ref[0, :, :, 0] = (s_t_ref[0, :, :, 0] + emb7[None, None]
                               + spat_pad_ref[...][:, :, None, :])


def kernel(s_t_x, sp_x, t_x, st_x, months, patch_size, input_res, pos_embed_p,
           month_tab, s_t_channel_embed, sp_channel_embed, t_channel_embed,
           st_channel_embed):
    b, h, w, t, g7, e = s_t_x.shape
    d = _D
    gsd = (jnp.asarray(input_res, jnp.float32)
           * jnp.asarray(patch_size, jnp.float32) / _BASE_GSD).reshape(1)

    grid = (b, t)
    full = lambda a: pl.BlockSpec(a.shape, lambda bi, ti: (0,) * a.ndim)
    in_specs = [
        pl.BlockSpec(months.shape, lambda bi, ti: (0, 0),
                     memory_space=pltpu.SMEM),
        pl.BlockSpec((1,), lambda bi, ti: (0,), memory_space=pltpu.SMEM),
        full(pos_embed_p), full(month_tab), full(s_t_channel_embed),
        full(sp_channel_embed), full(t_channel_embed), full(st_channel_embed),
        pl.BlockSpec((1, h, w, 1, g7, e), lambda bi, ti: (bi, 0, 0, ti, 0, 0)),
        pl.BlockSpec((1, h, w, 3, e), lambda bi, ti: (bi, 0, 0, 0, 0)),
        pl.BlockSpec((1, 1, 3, e), lambda bi, ti: (bi, ti, 0, 0)),
        pl.BlockSpec((1, 3, e), lambda bi, ti: (bi, 0, 0)),
    ]
    out_specs = [
        pl.BlockSpec((1, h, w, 1, g7, e), lambda bi, ti: (bi, 0, 0, ti, 0, 0)),
        pl.BlockSpec((1, h, w, 3, e), lambda bi, ti: (bi, 0, 0, 0, 0)),
        pl.BlockSpec((1, 1, 3, e), lambda bi, ti: (bi, ti, 0, 0)),
        pl.BlockSpec((1, 3, e), lambda bi, ti: (bi, 0, 0)),
    ]
    out_shapes = [
        jax.ShapeDtypeStruct(s_t_x.shape, jnp.float32),
        jax.ShapeDtypeStruct(sp_x.shape, jnp.float32),
        jax.ShapeDtypeStruct(t_x.shape, jnp.float32),
        jax.ShapeDtypeStruct(st_x.shape, jnp.float32),
    ]
    outs = pl.pallas_call(
        _tc_body,
        grid=grid,
        in_specs=in_specs,
        out_specs=out_specs,
        out_shape=out_shapes,
        scratch_shapes=[pltpu.VMEM((h, w, 4 * d), jnp.float32)],
        compiler_params=pltpu.CompilerParams(
            dimension_semantics=("arbitrary", "arbitrary")),
    )(months, gsd, pos_embed_p, month_tab, s_t_channel_embed,
      sp_channel_embed, t_channel_embed, st_channel_embed,
      s_t_x, sp_x, t_x, st_x)
    return tuple(outs)


# speedup vs baseline: 1.1088x; 1.1088x over previous
import functools
import math

import jax
import jax.numpy as jnp
from jax.experimental import pallas as pl
from jax.experimental.pallas import tpu as pltpu

_BASE_GSD = 10.0
_D = 192
_LN1E4_OVER = math.log(10000.0) / (_D // 4)
_TBLK = 4


def _tc_body(months_ref, gsd_ref, pos_ref, mtab_ref, ch7_ref, ch_sp_ref,
             ch_t_ref, ch_st_ref, s_t_ref, sp_ref, t_ref, st_ref,
             s_t_out_ref, sp_out_ref, t_out_ref, st_out_ref, spat_pad_ref):
    b = pl.program_id(0)
    tt = pl.program_id(1)
    h, w = 8, 8
    d = _D
    tb = _TBLK

    @pl.when(tt == 0)
    def _spatial_and_small():
        res = gsd_ref[0]
        wc = jax.lax.broadcasted_iota(jnp.int32, (h, w, d), 1).astype(jnp.float32)
        hc = jax.lax.broadcasted_iota(jnp.int32, (h, w, d), 0).astype(jnp.float32)
        col = jax.lax.broadcasted_iota(jnp.int32, (h, w, d), 2)
        p = jnp.where(col < d // 2, wc, hc) * res
        k = col % (d // 2)
        kk = (k % (d // 4)).astype(jnp.float32)
        omega = jnp.exp(kk * (-_LN1E4_OVER))
        phase = jnp.where(k < d // 4, 0.0, 0.5 * jnp.pi).astype(jnp.float32)
        spatial = jnp.sin(p * omega + phase)
        spat_pad_ref[...] = jnp.concatenate(
            [jnp.zeros((h, w, 3 * d), jnp.float32), spatial], axis=-1)

        sp_row = jnp.concatenate(
            [ch_sp_ref[...], jnp.zeros((3, 3 * d), jnp.float32)], axis=-1)
        sp_out_ref[0] = (sp_ref[0] + sp_row[None, None]
                         + spat_pad_ref[...][:, :, None, :])

        st_row = jnp.concatenate(
            [ch_st_ref[...], jnp.zeros((3, 3 * d), jnp.float32)], axis=-1)
        st_out_ref[0] = st_ref[0] + st_row

    pos4 = pos_ref[0]
    mon4 = jnp.concatenate(
        [mtab_ref[pl.ds(months_ref[b, tb * tt + i], 1), :] for i in range(tb)],
        axis=0)

    emb_t = jnp.concatenate(
        [jnp.broadcast_to(ch_t_ref[...][None], (tb, 3, d)),
         jnp.broadcast_to(pos4[:, None, :], (tb, 3, d)),
         jnp.broadcast_to(mon4[:, None, :], (tb, 3, d)),
         jnp.zeros((tb, 3, d), jnp.float32)], axis=-1)
    t_out_ref[0] = t_ref[0] + emb_t

    emb7 = jnp.concatenate(
        [jnp.broadcast_to(ch7_ref[...][None], (tb, 7, d)),
         jnp.broadcast_to(pos4[:, None, :], (tb, 7, d)),
         jnp.broadcast_to(mon4[:, None, :], (tb, 7, d)),
         jnp.zeros((tb, 7, d), jnp.float32)], axis=-1)
    s_t_out_ref[0] = (s_t_ref[0] + emb7[None, None]
                      + spat_pad_ref[...][:, :, None, None, :])


def kernel(s_t_x, sp_x, t_x, st_x, months, patch_size, input_res, pos_embed_p,
           month_tab, s_t_channel_embed, sp_channel_embed, t_channel_embed,
           st_channel_embed):
    b, h, w, t, g7, e = s_t_x.shape
    gsd = (jnp.asarray(input_res, jnp.float32)
           * jnp.asarray(patch_size, jnp.float32) / _BASE_GSD).reshape(1)

    grid = (b, t // _TBLK)
    full = lambda a: pl.BlockSpec(a.shape, lambda bi, ti: (0,) * a.ndim)
    in_specs = [
        pl.BlockSpec(months.shape, lambda bi, ti: (0, 0),
                     memory_space=pltpu.SMEM),
        pl.BlockSpec((1,), lambda bi, ti: (0,), memory_space=pltpu.SMEM),
        pl.BlockSpec((1, _TBLK, e // 4), lambda bi, ti: (ti, 0, 0)),
        full(month_tab), full(s_t_channel_embed),
        full(sp_channel_embed), full(t_channel_embed), full(st_channel_embed),
        pl.BlockSpec((1, h, w, _TBLK, g7, e),
                     lambda bi, ti: (bi, 0, 0, ti, 0, 0)),
        pl.BlockSpec((1, h, w, 3, e), lambda bi, ti: (bi, 0, 0, 0, 0)),
        pl.BlockSpec((1, _TBLK, 3, e), lambda bi, ti: (bi, ti, 0, 0)),
        pl.BlockSpec((1, 3, e), lambda bi, ti: (bi, 0, 0)),
    ]
    out_specs = [
        pl.BlockSpec((1, h, w, _TBLK, g7, e),
                     lambda bi, ti: (bi, 0, 0, ti, 0, 0)),
        pl.BlockSpec((1, h, w, 3, e), lambda bi, ti: (bi, 0, 0, 0, 0)),
        pl.BlockSpec((1, _TBLK, 3, e), lambda bi, ti: (bi, ti, 0, 0)),
        pl.BlockSpec((1, 3, e), lambda bi, ti: (bi, 0, 0)),
    ]
    out_shapes = [
        jax.ShapeDtypeStruct(s_t_x.shape, jnp.float32),
        jax.ShapeDtypeStruct(sp_x.shape, jnp.float32),
        jax.ShapeDtypeStruct(t_x.shape, jnp.float32),
        jax.ShapeDtypeStruct(st_x.shape, jnp.float32),
    ]
    outs = pl.pallas_call(
        _tc_body,
        grid=grid,
        in_specs=in_specs,
        out_specs=out_specs,
        out_shape=out_shapes,
        scratch_shapes=[pltpu.VMEM((h, w, 4 * _D), jnp.float32)],
        compiler_params=pltpu.CompilerParams(
            dimension_semantics=("arbitrary", "arbitrary")),
    )(months, gsd, pos_embed_p[:t].reshape(t // _TBLK, _TBLK, e // 4),
      month_tab, s_t_channel_embed,
      sp_channel_embed, t_channel_embed, st_channel_embed,
      s_t_x, sp_x, t_x, st_x)
    return tuple(outs)
